# Initial kernel scaffold; baseline (speedup 1.0000x reference)
import jax
import jax.numpy as jnp
from jax.experimental import pallas as pl


def kernel(X, force):
    raise NotImplementedError("write your pallas kernel here")


# speedup vs baseline: 66.9387x; 66.9387x over previous
import functools

import jax
import jax.numpy as jnp
from jax import lax
from jax.experimental import pallas as pl
from jax.experimental.pallas import tpu as pltpu
from jax.experimental.pallas import tpu_sc as plsc

_N = 256
_DT = 0.05
_DEPTH = 8
_BATCH = 262144
_L = 16
_NC = 2
_NS = 16
_NW = _NC * _NS
_BPW = _BATCH // _NW
_UNROLL = 4
_ITERS = _BPW // (_L * _UNROLL)

_INV_STEP = 2.56
_STEP = 0.390625
_DTB = 0.050048828125


def _rnbf16(a):
    u = plsc.bitcast(a, jnp.int32)
    u = (u + 0x7FFF + ((u >> 16) & 1)) & jnp.int32(-65536)
    return plsc.bitcast(u, jnp.float32)


def _nearest_idx(y):
    q = y * jnp.float32(_INV_STEP) + jnp.float32(0.5)
    q = q.astype(jnp.int32)
    q = jnp.minimum(jnp.maximum(q, 1), _N - 2)
    s = jnp.float32(_STEP)
    dm = jnp.abs(y - (q - 1).astype(jnp.float32) * s)
    d0 = jnp.abs(y - q.astype(jnp.float32) * s)
    dp = jnp.abs(y - (q + 1).astype(jnp.float32) * s)
    idx = q - 1
    best = dm
    sel = d0 < best
    idx = jnp.where(sel, q, idx)
    best = jnp.where(sel, d0, best)
    idx = jnp.where(dp < best, q + 1, idx)
    return idx


def _sc_body(x_hbm, f_hbm, out_hbm, buf, ftab):
    wid = lax.axis_index("s") * _NC + lax.axis_index("c")
    base = wid * (_BPW * 2)
    pltpu.sync_copy(f_hbm, ftab)
    pltpu.sync_copy(x_hbm.at[pl.ds(base, _BPW * 2)], buf)

    lane2 = lax.iota(jnp.int32, _L) * 2
    dt = jnp.float32(_DT)

    def step(i, carry):
        offs = [i * (2 * _L * _UNROLL) + u * (2 * _L) + lane2
                for u in range(_UNROLL)]
        xs = [plsc.load_gather(buf, [o]) for o in offs]
        vs = [plsc.load_gather(buf, [o + 1]) for o in offs]
        vs = [_rnbf16(v) for v in vs]
        xs = [_rnbf16(x) + v * jnp.float32(_DTB) for x, v in zip(xs, vs)]
        for _ in range(_DEPTH):
            idxs = [_nearest_idx(x * jnp.float32(100.0)) for x in xs]
            fs = [plsc.load_gather(ftab, [idx]) for idx in idxs]
            vs = [v + dt * f for v, f in zip(vs, fs)]
            xs = [x + v * dt for x, v in zip(xs, vs)]
        for o, x, v in zip(offs, xs, vs):
            plsc.store_scatter(buf, [o], x)
            plsc.store_scatter(buf, [o + 1], v)
        return carry

    lax.fori_loop(0, _ITERS, step, 0)
    pltpu.sync_copy(buf, out_hbm.at[pl.ds(base, _BPW * 2)])


_mesh = plsc.VectorSubcoreMesh(core_axis_name="c", subcore_axis_name="s")

_sc_kernel = functools.partial(
    pl.kernel,
    mesh=_mesh,
    out_type=jax.ShapeDtypeStruct((_BATCH * 2,), jnp.float32),
    scratch_types=[
        pltpu.VMEM((_BPW * 2,), jnp.float32),
        pltpu.VMEM((_N,), jnp.float32),
    ],
    compiler_params=pltpu.CompilerParams(needs_layout_passes=False),
)(_sc_body)


@jax.jit
def kernel(X, force):
    out = _sc_kernel(X.reshape(-1), force)
    return out.reshape(_BATCH, 2)


# speedup vs baseline: 68.3756x; 1.0215x over previous
import functools

import jax
import jax.numpy as jnp
from jax import lax
from jax.experimental import pallas as pl
from jax.experimental.pallas import tpu as pltpu
from jax.experimental.pallas import tpu_sc as plsc

_N = 256
_DT = 0.05
_DEPTH = 8
_BATCH = 262144
_L = 16
_NC = 2
_NS = 16
_NW = _NC * _NS
_BPW = _BATCH // _NW
_UNROLL = 4
_ITERS = _BPW // (_L * _UNROLL)

_INV_STEP = 2.56
_STEP = 0.390625
_DTB = 0.050048828125


def _rnbf16(a):
    u = plsc.bitcast(a, jnp.int32)
    u = (u + 0x7FFF + ((u >> 16) & 1)) & jnp.int32(-65536)
    return plsc.bitcast(u, jnp.float32)


def _nearest_idx(y):
    q = (y * jnp.float32(_INV_STEP) + jnp.float32(0.5)).astype(jnp.int32)
    r = y - q.astype(jnp.float32) * jnp.float32(_STEP)
    half = jnp.float32(_STEP * 0.5)
    q = q + jnp.where(r > half, 1, 0) - jnp.where(r <= -half, 1, 0)
    return jnp.minimum(q, _N - 1)


def _sc_body(x_hbm, f_hbm, out_hbm, buf, ftab):
    wid = lax.axis_index("s") * _NC + lax.axis_index("c")
    base = wid * (_BPW * 2)
    pltpu.sync_copy(f_hbm, ftab)
    pltpu.sync_copy(x_hbm.at[pl.ds(base, _BPW * 2)], buf)

    lane2 = lax.iota(jnp.int32, _L) * 2
    dt = jnp.float32(_DT)

    @plsc.parallel_loop(0, _BPW // _L, unroll=_UNROLL)
    def _loop(i):
        o = i * (2 * _L) + lane2
        x = plsc.load_gather(buf, [o])
        v = plsc.load_gather(buf, [o + 1])
        v = _rnbf16(v)
        x = _rnbf16(x) + v * jnp.float32(_DTB)
        for _ in range(_DEPTH):
            idx = _nearest_idx(x * jnp.float32(100.0))
            f = plsc.load_gather(ftab, [idx])
            v = v + dt * f
            x = x + v * dt
        plsc.store_scatter(buf, [o], x)
        plsc.store_scatter(buf, [o + 1], v)

    pltpu.sync_copy(buf, out_hbm.at[pl.ds(base, _BPW * 2)])


_mesh = plsc.VectorSubcoreMesh(core_axis_name="c", subcore_axis_name="s")

_sc_kernel = functools.partial(
    pl.kernel,
    mesh=_mesh,
    out_type=jax.ShapeDtypeStruct((_BATCH * 2,), jnp.float32),
    scratch_types=[
        pltpu.VMEM((_BPW * 2,), jnp.float32),
        pltpu.VMEM((_N,), jnp.float32),
    ],
    compiler_params=pltpu.CompilerParams(needs_layout_passes=False),
)(_sc_body)


@jax.jit
def kernel(X, force):
    out = _sc_kernel(X.reshape(-1), force)
    return out.reshape(_BATCH, 2)


# speedup vs baseline: 620.3532x; 9.0727x over previous
import functools

import jax
import jax.numpy as jnp
from jax import lax
from jax.experimental import pallas as pl
from jax.experimental.pallas import tpu as pltpu
from jax.experimental.pallas import tpu_sc as plsc

_N = 256
_DT = 0.05
_DEPTH = 8
_BATCH = 262144
_L = 16
_NC = 2
_NS = 16
_NW = _NC * _NS
_BPW = _BATCH // _NW
_UNROLL = 4
_ITERS = _BPW // (_L * _UNROLL)

_INV_STEP = 2.56
_STEP = 0.390625
_DTB = 0.050048828125


def _rnbf16(a):
    u = plsc.bitcast(a, jnp.int32)
    u = (u + 0x7FFF + ((u >> 16) & 1)) & jnp.int32(-65536)
    return plsc.bitcast(u, jnp.float32)


def _nearest_idx(y):
    q = (y * jnp.float32(_INV_STEP) + jnp.float32(0.5)).astype(jnp.int32)
    r = y - q.astype(jnp.float32) * jnp.float32(_STEP)
    half = jnp.float32(_STEP * 0.5)
    q = q + jnp.where(r > half, 1, 0) - jnp.where(r <= -half, 1, 0)
    return jnp.minimum(q, _N - 1)


def _sc_body(x_hbm, f_hbm, out_hbm, buf, ftab):
    wid = lax.axis_index("s") * _NC + lax.axis_index("c")
    base = wid * (_BPW * 2)
    pltpu.sync_copy(f_hbm, ftab)
    pltpu.sync_copy(x_hbm.at[pl.ds(base, _BPW * 2)], buf)

    dt = jnp.float32(_DT)

    @plsc.parallel_loop(0, _BPW // _L, unroll=_UNROLL)
    def _loop(i):
        off = (i >> 3) * 256 + (i & 7) * _L
        x = buf[pl.ds(off, _L)]
        v = buf[pl.ds(off + 128, _L)]
        v = _rnbf16(v)
        x = _rnbf16(x) + v * jnp.float32(_DTB)
        for _ in range(_DEPTH):
            idx = _nearest_idx(x * jnp.float32(100.0))
            f = plsc.load_gather(ftab, [idx])
            v = v + dt * f
            x = x + v * dt
        buf[pl.ds(off, _L)] = x
        buf[pl.ds(off + 128, _L)] = v

    pltpu.sync_copy(buf, out_hbm.at[pl.ds(base, _BPW * 2)])


_mesh = plsc.VectorSubcoreMesh(core_axis_name="c", subcore_axis_name="s")

_sc_kernel = functools.partial(
    pl.kernel,
    mesh=_mesh,
    out_type=jax.ShapeDtypeStruct((_BATCH * 2,), jnp.float32),
    scratch_types=[
        pltpu.VMEM((_BPW * 2,), jnp.float32),
        pltpu.VMEM((_N,), jnp.float32),
    ],
    compiler_params=pltpu.CompilerParams(needs_layout_passes=False),
)(_sc_body)


@jax.jit
def kernel(X, force):
    xf = X.reshape(_BATCH // 128, 128, 2).transpose(0, 2, 1).reshape(-1)
    out = _sc_kernel(xf, force)
    return (out.reshape(_BATCH // 128, 2, 128)
               .transpose(0, 2, 1).reshape(_BATCH, 2))
